# baseline (speedup 1.0000x reference)
import functools

import jax
import jax.numpy as jnp
from jax import lax
from jax.experimental import pallas as pl
from jax.experimental.pallas import tpu as pltpu
from jax.experimental.pallas import tpu_sc as plsc

VOCAB_N = 1000000
HID = 64
B_N = 4096
SEQ_N = 50

NC, NS, L = 2, 16, 16
NW = NC * NS
RW = B_N // NW
CHUNK = 16
NCHUNK = RW // CHUNK
GROWS = CHUNK * SEQ_N
GSTEP = 128
NG = (GROWS + GSTEP - 1) // GSTEP
NVH = HID // L


def _body(idx_hbm, table_hbm, out_hbm, idx_v, rows_v, outb_v, sem):
    wid = lax.axis_index("s") * NC + lax.axis_index("c")
    base_row = wid * RW
    pltpu.sync_copy(idx_hbm.at[pl.ds(wid * (RW * SEQ_N), RW * SEQ_N)],
                    idx_v.at[pl.ds(0, RW * SEQ_N)])
    lane = lax.iota(jnp.int32, L)

    for c in range(NCHUNK):
        goff = c * GROWS
        cps = []
        for g in range(NG):
            n = min(GSTEP, GROWS - g * GSTEP)
            cps.append(pltpu.async_copy(
                table_hbm.at[idx_v.at[pl.ds(goff + g * GSTEP, n)]],
                rows_v.at[pl.ds(g * GSTEP, n)], sem))
        for cp in cps:
            cp.wait()

        def rbody(r, carry):
            accs = [jnp.zeros((L,), jnp.float32) for _ in range(NVH)]
            for t in range(SEQ_N):
                for j in range(NVH):
                    accs[j] = accs[j] + rows_v[r * SEQ_N + t, pl.ds(j * L, L)]
            tbase = goff + r * SEQ_N
            ones = jnp.zeros((L,), jnp.float32)
            for k in range(3):
                tok = idx_v[pl.ds(tbase + k * L, L)]
                ones = ones + jnp.where(tok != 0, 1.0, 0.0).astype(jnp.float32)
            tok = idx_v[pl.ds(tbase + 3 * L, L)]
            ones = ones + jnp.where((tok != 0) & (lane < SEQ_N - 3 * L),
                                    1.0, 0.0).astype(jnp.float32)
            cnt = jnp.full((L,), jnp.sum(ones), jnp.float32)
            for j in range(NVH):
                outb_v[r, pl.ds(j * L, L)] = accs[j] / cnt
            return carry

        lax.fori_loop(0, CHUNK, rbody, 0)
        pltpu.sync_copy(outb_v, out_hbm.at[pl.ds(base_row + c * CHUNK, CHUNK)])


_sc_call = functools.partial(
    pl.kernel,
    out_type=jax.ShapeDtypeStruct((B_N, HID), jnp.float32),
    mesh=plsc.VectorSubcoreMesh(
        core_axis_name="c", subcore_axis_name="s",
        num_cores=NC, num_subcores=NS),
    compiler_params=pltpu.CompilerParams(
        needs_layout_passes=False, use_tc_tiling_on_sc=False),
    scratch_types=[
        pltpu.VMEM((RW * SEQ_N + L,), jnp.int32),
        pltpu.VMEM((GROWS, HID), jnp.float32),
        pltpu.VMEM((CHUNK, HID), jnp.float32),
        pltpu.SemaphoreType.DMA,
    ],
)(_body)


def kernel(title_tok, word_embeddings):
    idx_flat = title_tok.reshape(-1).astype(jnp.int32)
    return _sc_call(idx_flat, word_embeddings)


# speedup vs baseline: 1.2805x; 1.2805x over previous
import functools

import jax
import jax.numpy as jnp
from jax import lax
from jax.experimental import pallas as pl
from jax.experimental.pallas import tpu as pltpu
from jax.experimental.pallas import tpu_sc as plsc

VOCAB_N = 1000000
HID = 64
B_N = 4096
SEQ_N = 50

NC, NS, L = 2, 16, 16
NW = NC * NS
RW = B_N // NW
CHUNK = 16
NCHUNK = RW // CHUNK
GROWS = CHUNK * SEQ_N
GSTEP = 128
NG = (GROWS + GSTEP - 1) // GSTEP
NVH = HID // L

TQ = 1024
HBLK = 488
HALFP = HBLK * TQ
NPACK = 489
OUTR = NPACK * TQ
VPAD = 2 * OUTR


def _pack_body(lo_ref, hi_ref, out_ref):
    out_ref[:, 0:HID] = lo_ref[...].T
    out_ref[:, HID:2 * HID] = hi_ref[...].T


def _pack_table(tab_t):
    return pl.pallas_call(
        _pack_body,
        grid=(NPACK,),
        in_specs=[
            pl.BlockSpec((HID, TQ), lambda i: (0, i)),
            pl.BlockSpec((HID, TQ), lambda i: (0, HBLK + i)),
        ],
        out_specs=pl.BlockSpec((TQ, 2 * HID), lambda i: (i, 0)),
        out_shape=jax.ShapeDtypeStruct((OUTR, 2 * HID), jnp.float32),
    )(tab_t, tab_t)


def _body(idx_hbm, table_hbm, out_hbm, idx_v, idx2_v, rows_v, outb_v, sem):
    wid = lax.axis_index("s") * NC + lax.axis_index("c")
    base_row = wid * RW
    nidx = RW * SEQ_N
    pltpu.sync_copy(idx_hbm.at[pl.ds(wid * nidx, nidx)],
                    idx_v.at[pl.ds(0, nidx)])
    def mbody(v, carry):
        tok = idx_v[pl.ds(v * L, L)]
        idx2_v[pl.ds(v * L, L)] = 2 * tok - jnp.where(
            tok >= HALFP, 2 * HALFP - 1, 0).astype(jnp.int32)
        return carry

    lax.fori_loop(0, nidx // L, mbody, 0)

    lane = lax.iota(jnp.int32, L)

    for c in range(NCHUNK):
        goff = c * GROWS
        cps = []
        for g in range(NG):
            n = min(GSTEP, GROWS - g * GSTEP)
            cps.append(pltpu.async_copy(
                table_hbm.at[idx2_v.at[pl.ds(goff + g * GSTEP, n)]],
                rows_v.at[pl.ds(g * GSTEP, n)], sem))
        for cp in cps:
            cp.wait()

        def rbody(r, carry):
            accs = [jnp.zeros((L,), jnp.float32) for _ in range(NVH)]
            for t in range(SEQ_N):
                for j in range(NVH):
                    accs[j] = accs[j] + rows_v[r * SEQ_N + t, pl.ds(j * L, L)]
            tbase = goff + r * SEQ_N
            ones = jnp.zeros((L,), jnp.float32)
            for k in range(3):
                tok = idx_v[pl.ds(tbase + k * L, L)]
                ones = ones + jnp.where(tok != 0, 1.0, 0.0).astype(jnp.float32)
            tok = idx_v[pl.ds(tbase + 3 * L, L)]
            ones = ones + jnp.where((tok != 0) & (lane < SEQ_N - 3 * L),
                                    1.0, 0.0).astype(jnp.float32)
            cnt = jnp.full((L,), jnp.sum(ones), jnp.float32)
            for j in range(NVH):
                outb_v[r, pl.ds(j * L, L)] = accs[j] / cnt
            return carry

        lax.fori_loop(0, CHUNK, rbody, 0)
        pltpu.sync_copy(outb_v, out_hbm.at[pl.ds(base_row + c * CHUNK, CHUNK)])


_sc_call = functools.partial(
    pl.kernel,
    out_type=jax.ShapeDtypeStruct((B_N, HID), jnp.float32),
    mesh=plsc.VectorSubcoreMesh(
        core_axis_name="c", subcore_axis_name="s",
        num_cores=NC, num_subcores=NS),
    compiler_params=pltpu.CompilerParams(
        needs_layout_passes=False, use_tc_tiling_on_sc=False),
    scratch_types=[
        pltpu.VMEM((RW * SEQ_N + L,), jnp.int32),
        pltpu.VMEM((RW * SEQ_N,), jnp.int32),
        pltpu.VMEM((GROWS, HID), jnp.float32),
        pltpu.VMEM((CHUNK, HID), jnp.float32),
        pltpu.SemaphoreType.DMA,
    ],
)(_body)


def kernel(title_tok, word_embeddings):
    idx_flat = title_tok.reshape(-1).astype(jnp.int32)
    packed = _pack_table(word_embeddings.T)
    return _sc_call(idx_flat, packed.reshape(VPAD, HID))


# speedup vs baseline: 1.6755x; 1.3084x over previous
import functools

import jax
import jax.numpy as jnp
from jax import lax
from jax.experimental import pallas as pl
from jax.experimental.pallas import tpu as pltpu
from jax.experimental.pallas import tpu_sc as plsc

VOCAB_N = 1000000
HID = 64
B_N = 4096
SEQ_N = 50

NC, NS, L = 2, 16, 16
NW = NC * NS
RW = B_N // NW
CHUNK = 16
NCHUNK = RW // CHUNK
GROWS = CHUNK * SEQ_N
GSTEP = 128
NG = (GROWS + GSTEP - 1) // GSTEP
NVH = HID // L

TQ = 2048
HBLK = 244
HALFP = HBLK * TQ
NPACK = 245
OUTR = NPACK * TQ
VPAD = 2 * OUTR


def _pack_body(lo_ref, hi_ref, out_ref):
    r = lax.broadcasted_iota(jnp.int32, (HID, HID), 0)
    c = lax.broadcasted_iota(jnp.int32, (HID, HID), 1)
    eye = jnp.where(r == c, 1.0, 0.0).astype(jnp.float32)
    dn = (((0,), (0,)), ((), ()))
    out_ref[:, 0:HID] = lax.dot_general(
        lo_ref[...], eye, dn, preferred_element_type=jnp.float32)
    out_ref[:, HID:2 * HID] = lax.dot_general(
        hi_ref[...], eye, dn, preferred_element_type=jnp.float32)


def _pack_table(tab_t):
    return pl.pallas_call(
        _pack_body,
        grid=(NPACK,),
        in_specs=[
            pl.BlockSpec((HID, TQ), lambda i: (0, i)),
            pl.BlockSpec((HID, TQ), lambda i: (0, HBLK + i)),
        ],
        out_specs=pl.BlockSpec((TQ, 2 * HID), lambda i: (i, 0)),
        out_shape=jax.ShapeDtypeStruct((OUTR, 2 * HID), jnp.float32),
    )(tab_t, tab_t)


def _body(idx_hbm, table_hbm, out_hbm, idx_v, idx2_v, rows_v, outb_v, sem):
    wid = lax.axis_index("s") * NC + lax.axis_index("c")
    base_row = wid * RW
    nidx = RW * SEQ_N
    pltpu.sync_copy(idx_hbm.at[pl.ds(wid * nidx, nidx)],
                    idx_v.at[pl.ds(0, nidx)])
    def mbody(v, carry):
        tok = idx_v[pl.ds(v * L, L)]
        idx2_v[pl.ds(v * L, L)] = 2 * tok - jnp.where(
            tok >= HALFP, 2 * HALFP - 1, 0).astype(jnp.int32)
        return carry

    lax.fori_loop(0, nidx // L, mbody, 0)

    lane = lax.iota(jnp.int32, L)

    for c in range(NCHUNK):
        goff = c * GROWS
        cps = []
        for g in range(NG):
            n = min(GSTEP, GROWS - g * GSTEP)
            cps.append(pltpu.async_copy(
                table_hbm.at[idx2_v.at[pl.ds(goff + g * GSTEP, n)]],
                rows_v.at[pl.ds(g * GSTEP, n)], sem))
        for cp in cps:
            cp.wait()

        def rbody(r, carry):
            accs = [jnp.zeros((L,), jnp.float32) for _ in range(NVH)]
            for t in range(SEQ_N):
                for j in range(NVH):
                    accs[j] = accs[j] + rows_v[r * SEQ_N + t, pl.ds(j * L, L)]
            tbase = goff + r * SEQ_N
            ones = jnp.zeros((L,), jnp.float32)
            for k in range(3):
                tok = idx_v[pl.ds(tbase + k * L, L)]
                ones = ones + jnp.where(tok != 0, 1.0, 0.0).astype(jnp.float32)
            tok = idx_v[pl.ds(tbase + 3 * L, L)]
            ones = ones + jnp.where((tok != 0) & (lane < SEQ_N - 3 * L),
                                    1.0, 0.0).astype(jnp.float32)
            cnt = jnp.full((L,), jnp.sum(ones), jnp.float32)
            for j in range(NVH):
                outb_v[r, pl.ds(j * L, L)] = accs[j] / cnt
            return carry

        lax.fori_loop(0, CHUNK, rbody, 0)
        pltpu.sync_copy(outb_v, out_hbm.at[pl.ds(base_row + c * CHUNK, CHUNK)])


_sc_call = functools.partial(
    pl.kernel,
    out_type=jax.ShapeDtypeStruct((B_N, HID), jnp.float32),
    mesh=plsc.VectorSubcoreMesh(
        core_axis_name="c", subcore_axis_name="s",
        num_cores=NC, num_subcores=NS),
    compiler_params=pltpu.CompilerParams(
        needs_layout_passes=False, use_tc_tiling_on_sc=False),
    scratch_types=[
        pltpu.VMEM((RW * SEQ_N + L,), jnp.int32),
        pltpu.VMEM((RW * SEQ_N,), jnp.int32),
        pltpu.VMEM((GROWS, HID), jnp.float32),
        pltpu.VMEM((CHUNK, HID), jnp.float32),
        pltpu.SemaphoreType.DMA,
    ],
)(_body)


def kernel(title_tok, word_embeddings):
    idx_flat = title_tok.reshape(-1).astype(jnp.int32)
    packed = _pack_table(word_embeddings.T)
    return _sc_call(idx_flat, packed.reshape(VPAD, HID))


# speedup vs baseline: 1.7539x; 1.0468x over previous
import functools

import jax
import jax.numpy as jnp
from jax import lax
from jax.experimental import pallas as pl
from jax.experimental.pallas import tpu as pltpu
from jax.experimental.pallas import tpu_sc as plsc

VOCAB_N = 1000000
HID = 64
B_N = 4096
SEQ_N = 50

NC, NS, L = 2, 16, 16
NW = NC * NS
RW = B_N // NW
CHUNK = 16
NCHUNK = RW // CHUNK
GROWS = CHUNK * SEQ_N
GSTEP = 128
NG = (GROWS + GSTEP - 1) // GSTEP
NVH = HID // L

TQ = 2048
HBLK = 244
HALFP = HBLK * TQ
NPACK = 245
OUTR = NPACK * TQ
VPAD = 2 * OUTR


def _pack_body(lo_ref, hi_ref, out_ref):
    r = lax.broadcasted_iota(jnp.int32, (HID, 2 * HID), 0)
    c = lax.broadcasted_iota(jnp.int32, (HID, 2 * HID), 1)
    e_lo = jnp.where(r == c, 1.0, 0.0).astype(jnp.float32)
    e_hi = jnp.where(r + HID == c, 1.0, 0.0).astype(jnp.float32)
    dn = (((0,), (0,)), ((), ()))
    out_ref[...] = (
        lax.dot_general(lo_ref[...], e_lo, dn,
                        preferred_element_type=jnp.float32)
        + lax.dot_general(hi_ref[...], e_hi, dn,
                          preferred_element_type=jnp.float32))


def _pack_table(tab_t):
    return pl.pallas_call(
        _pack_body,
        grid=(NPACK,),
        in_specs=[
            pl.BlockSpec((HID, TQ), lambda i: (0, i)),
            pl.BlockSpec((HID, TQ), lambda i: (0, HBLK + i)),
        ],
        out_specs=pl.BlockSpec((TQ, 2 * HID), lambda i: (i, 0)),
        out_shape=jax.ShapeDtypeStruct((OUTR, 2 * HID), jnp.float32),
        compiler_params=pltpu.CompilerParams(
            fuse_transposed_lhs_in_matmul=True),
    )(tab_t, tab_t)


def _body(idx_hbm, table_hbm, out_hbm, idx_v, idx2_v, rows_v, outb_v, sem):
    wid = lax.axis_index("s") * NC + lax.axis_index("c")
    base_row = wid * RW
    nidx = RW * SEQ_N
    pltpu.sync_copy(idx_hbm.at[pl.ds(wid * nidx, nidx)],
                    idx_v.at[pl.ds(0, nidx)])
    def mbody(v, carry):
        tok = idx_v[pl.ds(v * L, L)]
        idx2_v[pl.ds(v * L, L)] = 2 * tok - jnp.where(
            tok >= HALFP, 2 * HALFP - 1, 0).astype(jnp.int32)
        return carry

    lax.fori_loop(0, nidx // L, mbody, 0)

    lane = lax.iota(jnp.int32, L)

    for c in range(NCHUNK):
        goff = c * GROWS
        cps = []
        for g in range(NG):
            n = min(GSTEP, GROWS - g * GSTEP)
            cps.append(pltpu.async_copy(
                table_hbm.at[idx2_v.at[pl.ds(goff + g * GSTEP, n)]],
                rows_v.at[pl.ds(g * GSTEP, n)], sem))
        for cp in cps:
            cp.wait()

        def rbody(r, carry):
            accs = [jnp.zeros((L,), jnp.float32) for _ in range(NVH)]
            for t in range(SEQ_N):
                for j in range(NVH):
                    accs[j] = accs[j] + rows_v[r * SEQ_N + t, pl.ds(j * L, L)]
            tbase = goff + r * SEQ_N
            ones = jnp.zeros((L,), jnp.float32)
            for k in range(3):
                tok = idx_v[pl.ds(tbase + k * L, L)]
                ones = ones + jnp.where(tok != 0, 1.0, 0.0).astype(jnp.float32)
            tok = idx_v[pl.ds(tbase + 3 * L, L)]
            ones = ones + jnp.where((tok != 0) & (lane < SEQ_N - 3 * L),
                                    1.0, 0.0).astype(jnp.float32)
            cnt = jnp.full((L,), jnp.sum(ones), jnp.float32)
            for j in range(NVH):
                outb_v[r, pl.ds(j * L, L)] = accs[j] / cnt
            return carry

        lax.fori_loop(0, CHUNK, rbody, 0)
        pltpu.sync_copy(outb_v, out_hbm.at[pl.ds(base_row + c * CHUNK, CHUNK)])


_sc_call = functools.partial(
    pl.kernel,
    out_type=jax.ShapeDtypeStruct((B_N, HID), jnp.float32),
    mesh=plsc.VectorSubcoreMesh(
        core_axis_name="c", subcore_axis_name="s",
        num_cores=NC, num_subcores=NS),
    compiler_params=pltpu.CompilerParams(
        needs_layout_passes=False, use_tc_tiling_on_sc=False),
    scratch_types=[
        pltpu.VMEM((RW * SEQ_N + L,), jnp.int32),
        pltpu.VMEM((RW * SEQ_N,), jnp.int32),
        pltpu.VMEM((GROWS, HID), jnp.float32),
        pltpu.VMEM((CHUNK, HID), jnp.float32),
        pltpu.SemaphoreType.DMA,
    ],
)(_body)


def kernel(title_tok, word_embeddings):
    idx_flat = title_tok.reshape(-1).astype(jnp.int32)
    packed = _pack_table(word_embeddings.T)
    return _sc_call(idx_flat, packed.reshape(VPAD, HID))


# speedup vs baseline: 2.1877x; 1.2473x over previous
import functools

import jax
import jax.numpy as jnp
from jax import lax
from jax.experimental import pallas as pl
from jax.experimental.pallas import tpu as pltpu
from jax.experimental.pallas import tpu_sc as plsc

VOCAB_N = 1000000
HID = 64
B_N = 4096
SEQ_N = 50

NC, NS, L = 2, 16, 16
NW = NC * NS
RW = B_N // NW
CHUNK = 16
NCHUNK = RW // CHUNK
GROWS = CHUNK * SEQ_N
GSTEP = 128
NG = (GROWS + GSTEP - 1) // GSTEP
NVH = HID // L

TQ = 4096
HBLK = 122
HALFP = HBLK * TQ
NPACK = 123
OUTR = NPACK * TQ
VPAD = 2 * OUTR


def _pack_body(lo_ref, hi_ref, out_ref):
    r = lax.broadcasted_iota(jnp.int32, (HID, 2 * HID), 0)
    c = lax.broadcasted_iota(jnp.int32, (HID, 2 * HID), 1)
    e_lo = jnp.where(r == c, 1.0, 0.0).astype(jnp.float32)
    e_hi = jnp.where(r + HID == c, 1.0, 0.0).astype(jnp.float32)
    dn = (((0,), (0,)), ((), ()))
    out_ref[...] = (
        lax.dot_general(lo_ref[...], e_lo, dn,
                        preferred_element_type=jnp.float32)
        + lax.dot_general(hi_ref[...], e_hi, dn,
                          preferred_element_type=jnp.float32))


def _pack_table(tab_t):
    return pl.pallas_call(
        _pack_body,
        grid=(NPACK,),
        in_specs=[
            pl.BlockSpec((HID, TQ), lambda i: (0, i)),
            pl.BlockSpec((HID, TQ), lambda i: (0, HBLK + i)),
        ],
        out_specs=pl.BlockSpec((TQ, 2 * HID), lambda i: (i, 0)),
        out_shape=jax.ShapeDtypeStruct((OUTR, 2 * HID), jnp.float32),
        compiler_params=pltpu.CompilerParams(
            fuse_transposed_lhs_in_matmul=True),
    )(tab_t, tab_t)


def _remap_body(idx_hbm, idx2_hbm, idx_v, idx2_v):
    wid = lax.axis_index("s") * NC + lax.axis_index("c")
    nidx = RW * SEQ_N
    pltpu.sync_copy(idx_hbm.at[pl.ds(wid * nidx, nidx)], idx_v)

    def mbody(v, carry):
        tok = idx_v[pl.ds(v * L, L)]
        idx2_v[pl.ds(v * L, L)] = 2 * tok - jnp.where(
            tok >= HALFP, 2 * HALFP - 1, 0).astype(jnp.int32)
        return carry

    lax.fori_loop(0, nidx // L, mbody, 0)
    pltpu.sync_copy(idx2_v, idx2_hbm.at[pl.ds(wid * nidx, nidx)])


_sc_remap = functools.partial(
    pl.kernel,
    out_type=jax.ShapeDtypeStruct((B_N * SEQ_N,), jnp.int32),
    mesh=plsc.VectorSubcoreMesh(
        core_axis_name="c", subcore_axis_name="s",
        num_cores=NC, num_subcores=NS),
    compiler_params=pltpu.CompilerParams(
        needs_layout_passes=False, use_tc_tiling_on_sc=False),
    scratch_types=[
        pltpu.VMEM((RW * SEQ_N,), jnp.int32),
        pltpu.VMEM((RW * SEQ_N,), jnp.int32),
    ],
)(_remap_body)


def _pool_body(idx2_hbm, table_hbm, out_hbm, idx_v, rows_v, outb_v, sem):
    wid = lax.axis_index("s") * NC + lax.axis_index("c")
    base_row = wid * RW
    nidx = RW * SEQ_N
    pltpu.sync_copy(idx2_hbm.at[pl.ds(wid * nidx, nidx)],
                    idx_v.at[pl.ds(0, nidx)])

    lane = lax.iota(jnp.int32, L)

    for c in range(NCHUNK):
        goff = c * GROWS
        cps = []
        for g in range(NG):
            n = min(GSTEP, GROWS - g * GSTEP)
            cps.append(pltpu.async_copy(
                table_hbm.at[idx_v.at[pl.ds(goff + g * GSTEP, n)]],
                rows_v.at[pl.ds(g * GSTEP, n)], sem))
        for cp in cps:
            cp.wait()

        def rbody(r, carry):
            accs = [jnp.zeros((L,), jnp.float32) for _ in range(NVH)]
            for t in range(SEQ_N):
                for j in range(NVH):
                    accs[j] = accs[j] + rows_v[r * SEQ_N + t, pl.ds(j * L, L)]
            tbase = goff + r * SEQ_N
            ones = jnp.zeros((L,), jnp.float32)
            for k in range(3):
                tok = idx_v[pl.ds(tbase + k * L, L)]
                ones = ones + jnp.where(tok != 0, 1.0, 0.0).astype(jnp.float32)
            tok = idx_v[pl.ds(tbase + 3 * L, L)]
            ones = ones + jnp.where((tok != 0) & (lane < SEQ_N - 3 * L),
                                    1.0, 0.0).astype(jnp.float32)
            cnt = jnp.full((L,), jnp.sum(ones), jnp.float32)
            for j in range(NVH):
                outb_v[r, pl.ds(j * L, L)] = accs[j] / cnt
            return carry

        lax.fori_loop(0, CHUNK, rbody, 0)
        pltpu.sync_copy(outb_v, out_hbm.at[pl.ds(base_row + c * CHUNK, CHUNK)])


_sc_pool = functools.partial(
    pl.kernel,
    out_type=jax.ShapeDtypeStruct((B_N, HID), jnp.float32),
    mesh=plsc.VectorSubcoreMesh(
        core_axis_name="c", subcore_axis_name="s",
        num_cores=NC, num_subcores=NS),
    compiler_params=pltpu.CompilerParams(
        needs_layout_passes=False, use_tc_tiling_on_sc=False),
    scratch_types=[
        pltpu.VMEM((RW * SEQ_N + L,), jnp.int32),
        pltpu.VMEM((GROWS, HID), jnp.float32),
        pltpu.VMEM((CHUNK, HID), jnp.float32),
        pltpu.SemaphoreType.DMA,
    ],
)(_pool_body)


def kernel(title_tok, word_embeddings):
    idx_flat = title_tok.reshape(-1).astype(jnp.int32)
    idx2 = _sc_remap(idx_flat)
    packed = _pack_table(word_embeddings.T)
    return _sc_pool(idx2, packed.reshape(VPAD, HID))


# speedup vs baseline: 2.6716x; 1.2212x over previous
import functools

import jax
import jax.numpy as jnp
from jax import lax
from jax.experimental import pallas as pl
from jax.experimental.pallas import tpu as pltpu
from jax.experimental.pallas import tpu_sc as plsc

VOCAB_N = 1000000
HID = 64
B_N = 4096
SEQ_N = 50

NC, NS, L = 2, 16, 16
NW = NC * NS
RW = B_N // NW
CHUNK = 16
NCHUNK = RW // CHUNK
GROWS = CHUNK * SEQ_N
GSTEP = 128
NG = (GROWS + GSTEP - 1) // GSTEP
NVH = HID // L

TQ = 8192
HBLK = 61
HALFP = HBLK * TQ
NPACK = 62
OUTR = NPACK * TQ
VPAD = 2 * OUTR


def _pack_body(lo_ref, hi_ref, out_ref):
    r = lax.broadcasted_iota(jnp.int32, (HID, 2 * HID), 0)
    c = lax.broadcasted_iota(jnp.int32, (HID, 2 * HID), 1)
    e_lo = jnp.where(r == c, 1.0, 0.0).astype(jnp.float32)
    e_hi = jnp.where(r + HID == c, 1.0, 0.0).astype(jnp.float32)
    dn = (((0,), (0,)), ((), ()))
    out_ref[...] = (
        lax.dot_general(lo_ref[...], e_lo, dn,
                        preferred_element_type=jnp.float32)
        + lax.dot_general(hi_ref[...], e_hi, dn,
                          preferred_element_type=jnp.float32))


def _pack_table(tab_t):
    return pl.pallas_call(
        _pack_body,
        grid=(NPACK,),
        in_specs=[
            pl.BlockSpec((HID, TQ), lambda i: (0, i)),
            pl.BlockSpec((HID, TQ), lambda i: (0, HBLK + i)),
        ],
        out_specs=pl.BlockSpec((TQ, 2 * HID), lambda i: (i, 0)),
        out_shape=jax.ShapeDtypeStruct((OUTR, 2 * HID), jnp.float32),
        compiler_params=pltpu.CompilerParams(
            fuse_transposed_lhs_in_matmul=True),
    )(tab_t, tab_t)


def _remap_body(idx_hbm, idx2_hbm, idx_v, idx2_v):
    wid = lax.axis_index("s") * NC + lax.axis_index("c")
    nidx = RW * SEQ_N
    pltpu.sync_copy(idx_hbm.at[pl.ds(wid * nidx, nidx)], idx_v)

    def mbody(v, carry):
        tok = idx_v[pl.ds(v * L, L)]
        idx2_v[pl.ds(v * L, L)] = 2 * tok - jnp.where(
            tok >= HALFP, 2 * HALFP - 1, 0).astype(jnp.int32)
        return carry

    lax.fori_loop(0, nidx // L, mbody, 0)
    pltpu.sync_copy(idx2_v, idx2_hbm.at[pl.ds(wid * nidx, nidx)])


_sc_remap = functools.partial(
    pl.kernel,
    out_type=jax.ShapeDtypeStruct((B_N * SEQ_N,), jnp.int32),
    mesh=plsc.VectorSubcoreMesh(
        core_axis_name="c", subcore_axis_name="s",
        num_cores=NC, num_subcores=NS),
    compiler_params=pltpu.CompilerParams(
        needs_layout_passes=False, use_tc_tiling_on_sc=False),
    scratch_types=[
        pltpu.VMEM((RW * SEQ_N,), jnp.int32),
        pltpu.VMEM((RW * SEQ_N,), jnp.int32),
    ],
)(_remap_body)


def _pool_body(idx2_hbm, table_hbm, out_hbm, idx_v, rows0_v, rows1_v,
               outb_v, sem0, sem1):
    wid = lax.axis_index("s") * NC + lax.axis_index("c")
    base_row = wid * RW
    nidx = RW * SEQ_N
    pltpu.sync_copy(idx2_hbm.at[pl.ds(wid * nidx, nidx)],
                    idx_v.at[pl.ds(0, nidx)])

    lane = lax.iota(jnp.int32, L)
    bufs = (rows0_v, rows1_v)
    sems = (sem0, sem1)

    def fire(c):
        goff = c * GROWS
        buf, sem = bufs[c % 2], sems[c % 2]
        return [pltpu.async_copy(
            table_hbm.at[idx_v.at[pl.ds(goff + g * GSTEP,
                                        min(GSTEP, GROWS - g * GSTEP))]],
            buf.at[pl.ds(g * GSTEP, min(GSTEP, GROWS - g * GSTEP))], sem)
            for g in range(NG)]

    cps = fire(0)
    for c in range(NCHUNK):
        goff = c * GROWS
        rows_v = bufs[c % 2]
        for cp in cps:
            cp.wait()
        if c + 1 < NCHUNK:
            cps = fire(c + 1)

        def rbody(r, carry):
            accs = [jnp.zeros((L,), jnp.float32) for _ in range(NVH)]
            for t in range(SEQ_N):
                for j in range(NVH):
                    accs[j] = accs[j] + rows_v[r * SEQ_N + t, pl.ds(j * L, L)]
            tbase = goff + r * SEQ_N
            ones = jnp.zeros((L,), jnp.float32)
            for k in range(3):
                tok = idx_v[pl.ds(tbase + k * L, L)]
                ones = ones + jnp.where(tok != 0, 1.0, 0.0).astype(jnp.float32)
            tok = idx_v[pl.ds(tbase + 3 * L, L)]
            ones = ones + jnp.where((tok != 0) & (lane < SEQ_N - 3 * L),
                                    1.0, 0.0).astype(jnp.float32)
            cnt = jnp.full((L,), jnp.sum(ones), jnp.float32)
            for j in range(NVH):
                outb_v[r, pl.ds(j * L, L)] = accs[j] / cnt
            return carry

        lax.fori_loop(0, CHUNK, rbody, 0)
        pltpu.sync_copy(outb_v, out_hbm.at[pl.ds(base_row + c * CHUNK, CHUNK)])


_sc_pool = functools.partial(
    pl.kernel,
    out_type=jax.ShapeDtypeStruct((B_N, HID), jnp.float32),
    mesh=plsc.VectorSubcoreMesh(
        core_axis_name="c", subcore_axis_name="s",
        num_cores=NC, num_subcores=NS),
    compiler_params=pltpu.CompilerParams(
        needs_layout_passes=False, use_tc_tiling_on_sc=False),
    scratch_types=[
        pltpu.VMEM((RW * SEQ_N + L,), jnp.int32),
        pltpu.VMEM((GROWS, HID), jnp.float32),
        pltpu.VMEM((GROWS, HID), jnp.float32),
        pltpu.VMEM((CHUNK, HID), jnp.float32),
        pltpu.SemaphoreType.DMA,
        pltpu.SemaphoreType.DMA,
    ],
)(_pool_body)


def kernel(title_tok, word_embeddings):
    idx_flat = title_tok.reshape(-1).astype(jnp.int32)
    idx2 = _sc_remap(idx_flat)
    packed = _pack_table(word_embeddings.T)
    return _sc_pool(idx2, packed.reshape(VPAD, HID))
